# baseline (speedup 1.0000x reference)
import functools

import jax
import jax.numpy as jnp
from jax import lax
from jax.experimental import pallas as pl
from jax.experimental.pallas import tpu as pltpu
from jax.experimental.pallas import tpu_sc as plsc

NUM_CLASSES = 100000
DIM = 128
BATCH = 16384
NC = 2
NS = 16
NW = NC * NS
ROWS_PER_W = BATCH // NW
SUB = 128
NSUB = ROWS_PER_W // SUB
HIST_PER_TILE = 6272
HIST_PAD = NS * HIST_PER_TILE
Y_PER_TILE = BATCH // NS

_mesh = plsc.VectorSubcoreMesh(core_axis_name="c", subcore_axis_name="s")


@functools.partial(
    pl.kernel,
    out_type=jax.ShapeDtypeStruct((NW, 16), jnp.float32),
    mesh=_mesh,
    scratch_types=[
        pltpu.VMEM((HIST_PER_TILE,), jnp.float32),
        pltpu.VMEM((Y_PER_TILE,), jnp.float32),
        pltpu.VMEM((Y_PER_TILE,), jnp.int32),
        pltpu.VMEM((SUB,), jnp.int32),
        pltpu.VMEM((SUB,), jnp.float32),
        pltpu.VMEM((SUB,), jnp.float32),
        pltpu.VMEM((SUB, DIM), jnp.float32),
        pltpu.VMEM((SUB, DIM), jnp.float32),
        pltpu.VMEM((16,), jnp.float32),
        pltpu.VMEM_SHARED((HIST_PAD,), jnp.float32),
        pltpu.SemaphoreType.DMA,
    ],
)
def _center_loss_sc(y_hbm, hidden_hbm, centers_hbm, out_hbm,
                    zbuf, ones_v, ych, idx_c, cnt_c, inv_c, crows, hrows,
                    tv, hist, sem):
    cid = lax.axis_index("c")
    sid = lax.axis_index("s")
    wid = cid * NS + sid

    zeros16 = jnp.zeros((16,), jnp.float32)
    ones16 = jnp.ones((16,), jnp.float32)

    def fill_z(i, carry):
        zbuf[pl.ds(i * 16, 16)] = zeros16
        return carry

    lax.fori_loop(0, HIST_PER_TILE // 16, fill_z, 0)

    def fill_o(i, carry):
        ones_v[pl.ds(i * 16, 16)] = ones16
        return carry

    lax.fori_loop(0, Y_PER_TILE // 16, fill_o, 0)

    pltpu.sync_copy(zbuf, hist.at[pl.ds(sid * HIST_PER_TILE, HIST_PER_TILE)])
    pltpu.sync_copy(y_hbm.at[pl.ds(sid * Y_PER_TILE, Y_PER_TILE)], ych)
    plsc.subcore_barrier()
    pltpu.sync_copy(ones_v, hist.at[ych], add=True)
    plsc.subcore_barrier()

    base = wid * ROWS_PER_W
    total = jnp.zeros((16,), jnp.float32)
    for sub in range(NSUB):
        row0 = base + sub * SUB
        pltpu.sync_copy(y_hbm.at[pl.ds(row0, SUB)], idx_c)
        pltpu.sync_copy(hist.at[idx_c], cnt_c)
        pltpu.async_copy(centers_hbm.at[idx_c], crows, sem).wait()
        pltpu.sync_copy(hidden_hbm.at[pl.ds(row0, SUB)], hrows)
        for i in range(SUB // 16):
            c16 = cnt_c[pl.ds(i * 16, 16)]
            inv_c[pl.ds(i * 16, 16)] = 0.5 / (c16 + 1.0)

        def group_body(g, tot):
            inv16 = inv_c[pl.ds(g * 16, 16)]
            for rr in range(16):
                row = g * 16 + rr
                acc = zeros16
                for k in range(DIM // 16):
                    hv = hrows[row, pl.ds(k * 16, 16)]
                    cv = crows[row, pl.ds(k * 16, 16)]
                    d = hv - cv
                    acc = acc + d * d
                inv_r = lax.gather(
                    inv16, jnp.full((16, 1), rr, jnp.int32),
                    lax.GatherDimensionNumbers(
                        offset_dims=(), collapsed_slice_dims=(0,),
                        start_index_map=(0,)),
                    slice_sizes=(1,),
                    mode=lax.GatherScatterMode.PROMISE_IN_BOUNDS)
                tot = tot + acc * inv_r
            return tot

        total = lax.fori_loop(0, SUB // 16, group_body, total)

    tv[...] = total
    pltpu.sync_copy(tv, out_hbm.at[wid])


def kernel(y, hidden, centers):
    parts = _center_loss_sc(y.astype(jnp.int32), hidden, centers)
    return jnp.sum(parts)


# speedup vs baseline: 1.2055x; 1.2055x over previous
import functools

import jax
import jax.numpy as jnp
from jax import lax
from jax.experimental import pallas as pl
from jax.experimental.pallas import tpu as pltpu
from jax.experimental.pallas import tpu_sc as plsc

NUM_CLASSES = 100000
DIM = 128
BATCH = 16384
NC = 2
NS = 16
NW = NC * NS
ROWS_PER_W = BATCH // NW
SUB = 128
NSUB = ROWS_PER_W // SUB
NBUF = 2
HIST_PER_TILE = 6272
HIST_PAD = NS * HIST_PER_TILE
Y_PER_TILE = BATCH // NS

_mesh = plsc.VectorSubcoreMesh(core_axis_name="c", subcore_axis_name="s")


@functools.partial(
    pl.kernel,
    out_type=jax.ShapeDtypeStruct((NW, 16), jnp.float32),
    mesh=_mesh,
    scratch_types=[
        pltpu.VMEM((HIST_PER_TILE,), jnp.float32),
        pltpu.VMEM((Y_PER_TILE,), jnp.float32),
        pltpu.VMEM((Y_PER_TILE,), jnp.int32),
        pltpu.VMEM((ROWS_PER_W,), jnp.int32),
        pltpu.VMEM((ROWS_PER_W,), jnp.float32),
        pltpu.VMEM((ROWS_PER_W,), jnp.float32),
        pltpu.VMEM((NBUF, SUB, DIM), jnp.float32),
        pltpu.VMEM((NBUF, SUB, DIM), jnp.float32),
        pltpu.VMEM((16,), jnp.float32),
        pltpu.VMEM_SHARED((HIST_PAD,), jnp.float32),
        pltpu.SemaphoreType.DMA,
        pltpu.SemaphoreType.DMA,
    ],
)
def _center_loss_sc(y_hbm, hidden_hbm, centers_hbm, out_hbm,
                    zbuf, ones_v, ych, idx_v, cnt_v, inv_v, crows, hrows,
                    tv, hist, sem_c, sem_h):
    cid = lax.axis_index("c")
    sid = lax.axis_index("s")
    wid = cid * NS + sid
    base = wid * ROWS_PER_W

    zeros16 = jnp.zeros((16,), jnp.float32)
    ones16 = jnp.ones((16,), jnp.float32)

    pltpu.sync_copy(y_hbm.at[pl.ds(base, ROWS_PER_W)], idx_v)

    def start_chunk(t):
        buf = t % NBUF
        pltpu.async_copy(
            centers_hbm.at[idx_v.at[pl.ds(t * SUB, SUB)]], crows.at[buf], sem_c)
        pltpu.async_copy(
            hidden_hbm.at[pl.ds(base + t * SUB, SUB)], hrows.at[buf], sem_h)

    def wait_chunk(t):
        buf = t % NBUF
        pltpu.make_async_copy(
            centers_hbm.at[idx_v.at[pl.ds(t * SUB, SUB)]], crows.at[buf],
            sem_c).wait()
        pltpu.make_async_copy(
            hidden_hbm.at[pl.ds(base + t * SUB, SUB)], hrows.at[buf],
            sem_h).wait()

    for t in range(NBUF):
        start_chunk(t)

    def fill_z(i, carry):
        zbuf[pl.ds(i * 16, 16)] = zeros16
        return carry

    lax.fori_loop(0, HIST_PER_TILE // 16, fill_z, 0, unroll=8)

    def fill_o(i, carry):
        ones_v[pl.ds(i * 16, 16)] = ones16
        return carry

    lax.fori_loop(0, Y_PER_TILE // 16, fill_o, 0, unroll=8)

    pltpu.sync_copy(zbuf, hist.at[pl.ds(sid * HIST_PER_TILE, HIST_PER_TILE)])
    pltpu.sync_copy(y_hbm.at[pl.ds(sid * Y_PER_TILE, Y_PER_TILE)], ych)
    plsc.subcore_barrier()
    pltpu.sync_copy(ones_v, hist.at[ych], add=True)
    plsc.subcore_barrier()

    pltpu.sync_copy(hist.at[idx_v], cnt_v)

    def fill_inv(i, carry):
        c16 = cnt_v[pl.ds(i * 16, 16)]
        inv_v[pl.ds(i * 16, 16)] = 0.5 / (c16 + 1.0)
        return carry

    lax.fori_loop(0, ROWS_PER_W // 16, fill_inv, 0, unroll=8)

    total = zeros16
    for t in range(NSUB):
        wait_chunk(t)
        if t + NBUF < NSUB:
            start_chunk(t + NBUF)
        buf = t % NBUF
        cbuf = crows.at[buf]
        hbuf = hrows.at[buf]
        inv_base = t * SUB

        def group_body(g, tot):
            inv16 = inv_v[pl.ds(inv_base + g * 16, 16)]
            for rr in range(16):
                row = g * 16 + rr
                acc = zeros16
                for k in range(DIM // 16):
                    hv = hbuf[row, pl.ds(k * 16, 16)]
                    cv = cbuf[row, pl.ds(k * 16, 16)]
                    d = hv - cv
                    acc = acc + d * d
                inv_r = lax.gather(
                    inv16, jnp.full((16, 1), rr, jnp.int32),
                    lax.GatherDimensionNumbers(
                        offset_dims=(), collapsed_slice_dims=(0,),
                        start_index_map=(0,)),
                    slice_sizes=(1,),
                    mode=lax.GatherScatterMode.PROMISE_IN_BOUNDS)
                tot = tot + acc * inv_r
            return tot

        total = lax.fori_loop(0, SUB // 16, group_body, total)

    tv[...] = total
    pltpu.sync_copy(tv, out_hbm.at[wid])


def kernel(y, hidden, centers):
    parts = _center_loss_sc(y.astype(jnp.int32), hidden, centers)
    return jnp.sum(parts)


# speedup vs baseline: 1.2349x; 1.0244x over previous
import functools

import jax
import jax.numpy as jnp
from jax import lax
from jax.experimental import pallas as pl
from jax.experimental.pallas import tpu as pltpu
from jax.experimental.pallas import tpu_sc as plsc

NUM_CLASSES = 100000
DIM = 128
BATCH = 16384
NC = 2
NS = 16
NW = NC * NS
ROWS_PER_W = BATCH // NW
SUB = 128
NSUB = ROWS_PER_W // SUB
NBUF = 2
HIST_PER_TILE = 6272
HIST_PAD = NS * HIST_PER_TILE
Y_PER_TILE = BATCH // NS

_mesh = plsc.VectorSubcoreMesh(core_axis_name="c", subcore_axis_name="s")


@functools.partial(
    pl.kernel,
    out_type=jax.ShapeDtypeStruct((NW, 16), jnp.float32),
    mesh=_mesh,
    scratch_types=[
        pltpu.VMEM((HIST_PER_TILE,), jnp.float32),
        pltpu.VMEM((Y_PER_TILE,), jnp.float32),
        pltpu.VMEM((Y_PER_TILE,), jnp.int32),
        pltpu.VMEM((ROWS_PER_W,), jnp.int32),
        pltpu.VMEM((ROWS_PER_W,), jnp.float32),
        pltpu.VMEM((ROWS_PER_W,), jnp.float32),
        pltpu.VMEM((NBUF, SUB, DIM), jnp.float32),
        pltpu.VMEM((NBUF, SUB, DIM), jnp.float32),
        pltpu.VMEM((16,), jnp.float32),
        pltpu.VMEM_SHARED((HIST_PAD,), jnp.float32),
        pltpu.SemaphoreType.DMA,
        pltpu.SemaphoreType.DMA,
    ],
)
def _center_loss_sc(y_hbm, hidden_hbm, centers_hbm, out_hbm,
                    zbuf, ones_v, ych, idx_v, cnt_v, inv_v, crows, hrows,
                    tv, hist, sem_c, sem_h):
    cid = lax.axis_index("c")
    sid = lax.axis_index("s")
    wid = cid * NS + sid
    base = wid * ROWS_PER_W

    zeros16 = jnp.zeros((16,), jnp.float32)
    ones16 = jnp.ones((16,), jnp.float32)

    pltpu.sync_copy(y_hbm.at[pl.ds(base, ROWS_PER_W)], idx_v)

    def start_chunk(t):
        buf = t % NBUF
        pltpu.async_copy(
            centers_hbm.at[idx_v.at[pl.ds(t * SUB, SUB)]], crows.at[buf], sem_c)
        pltpu.async_copy(
            hidden_hbm.at[pl.ds(base + t * SUB, SUB)], hrows.at[buf], sem_h)

    def wait_chunk(t):
        buf = t % NBUF
        pltpu.make_async_copy(
            centers_hbm.at[idx_v.at[pl.ds(t * SUB, SUB)]], crows.at[buf],
            sem_c).wait()
        pltpu.make_async_copy(
            hidden_hbm.at[pl.ds(base + t * SUB, SUB)], hrows.at[buf],
            sem_h).wait()

    for t in range(NBUF):
        start_chunk(t)

    with jax.named_scope("fills"):
        def fill_z(i, carry):
            zbuf[pl.ds(i * 16, 16)] = zeros16
            return carry

        lax.fori_loop(0, HIST_PER_TILE // 16, fill_z, 0, unroll=8)

        def fill_o(i, carry):
            ones_v[pl.ds(i * 16, 16)] = ones16
            return carry

        lax.fori_loop(0, Y_PER_TILE // 16, fill_o, 0, unroll=8)

    with jax.named_scope("hist"):
        pltpu.sync_copy(zbuf, hist.at[pl.ds(sid * HIST_PER_TILE, HIST_PER_TILE)])
        pltpu.sync_copy(y_hbm.at[pl.ds(sid * Y_PER_TILE, Y_PER_TILE)], ych)
        plsc.subcore_barrier()
        pltpu.sync_copy(ones_v, hist.at[ych], add=True)
        plsc.subcore_barrier()

    with jax.named_scope("counts"):
        pltpu.sync_copy(hist.at[idx_v], cnt_v)

        def fill_inv(i, carry):
            c16 = cnt_v[pl.ds(i * 16, 16)]
            inv_v[pl.ds(i * 16, 16)] = 0.5 / (c16 + 1.0)
            return carry

        lax.fori_loop(0, ROWS_PER_W // 16, fill_inv, 0, unroll=8)

    total = zeros16
    for t in range(NSUB):
        with jax.named_scope(f"wait{t}"):
            wait_chunk(t)
        if t + NBUF < NSUB:
            start_chunk(t + NBUF)
        buf = t % NBUF
        cbuf = crows.at[buf]
        hbuf = hrows.at[buf]
        inv_base = t * SUB

        def group_body(g, tot):
            inv16 = inv_v[pl.ds(inv_base + g * 16, 16)]
            for rr in range(16):
                row = g * 16 + rr
                acc = zeros16
                for k in range(DIM // 16):
                    hv = hbuf[row, pl.ds(k * 16, 16)]
                    cv = cbuf[row, pl.ds(k * 16, 16)]
                    d = hv - cv
                    acc = acc + d * d
                inv_r = lax.gather(
                    inv16, jnp.full((16, 1), rr, jnp.int32),
                    lax.GatherDimensionNumbers(
                        offset_dims=(), collapsed_slice_dims=(0,),
                        start_index_map=(0,)),
                    slice_sizes=(1,),
                    mode=lax.GatherScatterMode.PROMISE_IN_BOUNDS)
                tot = tot + acc * inv_r
            return tot

        with jax.named_scope(f"compute{t}"):
            total = lax.fori_loop(0, SUB // 16, group_body, total)

    tv[...] = total
    pltpu.sync_copy(tv, out_hbm.at[wid])


def kernel(y, hidden, centers):
    parts = _center_loss_sc(y.astype(jnp.int32), hidden, centers)
    return jnp.sum(parts)
